# Initial kernel scaffold; baseline (speedup 1.0000x reference)
import functools

import jax
import jax.numpy as jnp
from jax import lax
from jax.experimental import pallas as pl
from jax.experimental.pallas import tpu as pltpu
from jax.experimental.pallas import tpu_sc as plsc

N = 50000
E = 800000
EMB = 64
HALF = 32
NSUB = 16
NCORE = 2
CHUNK = 80
GROUP = 5
ROWS_PER_SUB = N // NSUB
EROWS = E // CHUNK
SUB_EROWS = EROWS // NSUB
ITERS = SUB_EROWS // GROUP

NROWS_R = NCORE * N * HALF // 128
BLK_R = 1000
GRID_R = NROWS_R // BLK_R


def _sc_mesh():
    return plsc.VectorSubcoreMesh(core_axis_name="c", subcore_axis_name="s")


def _edge_pass_body(with_gather, *refs):
    if with_gather:
        (row3, col3, y3, zeros, out,
         idx_r_v, idx_c_v, rows_v, acc, sem_i, sem_g, sem_s) = refs
    else:
        (col3, zeros, ones, out,
         idx_r_v, idx_c_v, rows_v, acc, sem_i, sem_g, sem_s) = refs
    c = lax.axis_index("c")
    s = lax.axis_index("s")

    pltpu.sync_copy(zeros, acc.at[pl.ds(s * ROWS_PER_SUB, ROWS_PER_SUB)])
    if not with_gather:
        for j in range(GROUP):
            pltpu.sync_copy(ones, rows_v.at[j])
    plsc.subcore_barrier()

    def iter_body(g, carry):
        base = s * SUB_EROWS + g * GROUP
        if with_gather:
            cp_r = pltpu.async_copy(row3.at[pl.ds(base, GROUP)], idx_r_v, sem_i)
        cp_c = pltpu.async_copy(col3.at[pl.ds(base, GROUP)], idx_c_v, sem_i)
        if with_gather:
            cp_r.wait()
            cp_c.wait()
            gathers = [
                pltpu.async_copy(y3.at[c].at[idx_r_v.at[j]], rows_v.at[j], sem_g)
                for j in range(GROUP)
            ]
            for g_cp in gathers:
                g_cp.wait()
        else:
            cp_c.wait()
        scatters = [
            pltpu.async_copy(rows_v.at[j], acc.at[idx_c_v.at[j]], sem_s, add=True)
            for j in range(GROUP)
        ]
        for s_cp in scatters:
            s_cp.wait()
        return carry

    lax.fori_loop(0, ITERS, iter_body, 0)
    plsc.subcore_barrier()
    pltpu.sync_copy(
        acc.at[pl.ds(s * ROWS_PER_SUB, ROWS_PER_SUB)],
        out.at[c].at[pl.ds(s * ROWS_PER_SUB, ROWS_PER_SUB)],
    )


def _make_edge_pass(with_gather):
    return pl.kernel(
        functools.partial(_edge_pass_body, with_gather),
        out_type=jax.ShapeDtypeStruct((NCORE, N, HALF), jnp.float32),
        mesh=_sc_mesh(),
        scratch_types=[
            pltpu.VMEM((GROUP, CHUNK), jnp.int32),
            pltpu.VMEM((GROUP, CHUNK), jnp.int32),
            pltpu.VMEM((GROUP, CHUNK, HALF), jnp.float32),
            pltpu.VMEM_SHARED((N, HALF), jnp.float32),
            pltpu.SemaphoreType.DMA,
            pltpu.SemaphoreType.DMA,
            pltpu.SemaphoreType.DMA,
        ],
    )


_sc_scatter = _make_edge_pass(True)
_sc_deg = _make_edge_pass(False)


def _tc_prep_body(deg_ref, x_ref, dis_ref, y_ref):
    deg = deg_ref[...]
    dis = jnp.where(deg > 0.0, lax.rsqrt(deg), 0.0)
    dis_ref[...] = dis
    y_ref[...] = dis * x_ref[...]


def _tc_prep(deg_r, x_r):
    spec = pl.BlockSpec((BLK_R, 128), lambda i: (i, 0))
    return pl.pallas_call(
        _tc_prep_body,
        grid=(GRID_R,),
        in_specs=[spec, spec],
        out_specs=[spec, spec],
        out_shape=[jax.ShapeDtypeStruct((NROWS_R, 128), jnp.float32)] * 2,
    )(deg_r, x_r)


def _tc_scale_body(is_final, z_ref, dis_ref, s_ref, so_ref, y_ref):
    dis = dis_ref[...]
    xk = dis * z_ref[...]
    snew = s_ref[...] + xk
    so_ref[...] = snew * 0.25 if is_final else snew
    y_ref[...] = dis * xk


def _tc_scale(z_r, dis_r, s_r, is_final):
    spec = pl.BlockSpec((BLK_R, 128), lambda i: (i, 0))
    return pl.pallas_call(
        functools.partial(_tc_scale_body, is_final),
        grid=(GRID_R,),
        in_specs=[spec, spec, spec],
        out_specs=[spec, spec],
        out_shape=[jax.ShapeDtypeStruct((NROWS_R, 128), jnp.float32)] * 2,
    )(z_r, dis_r, s_r)


def kernel(edge_index, user_emb, item_emb):
    x0 = jnp.concatenate([user_emb, item_emb], axis=0)
    x0s = jnp.stack([x0[:, :HALF], x0[:, HALF:]])
    x0_r = x0s.reshape(NROWS_R, 128)
    row3 = edge_index[0].reshape(EROWS, CHUNK)
    col3 = edge_index[1].reshape(EROWS, CHUNK)
    zeros = jnp.zeros((ROWS_PER_SUB, HALF), jnp.float32)
    ones = jnp.ones((CHUNK, HALF), jnp.float32)

    deg3 = _sc_deg(col3, zeros, ones)
    dis_r, y_r = _tc_prep(deg3.reshape(NROWS_R, 128), x0_r)
    s_r = x0_r
    for k in range(3):
        z3 = _sc_scatter(row3, col3, y_r.reshape(NCORE, N, HALF), zeros)
        s_r, y_r = _tc_scale(z3.reshape(NROWS_R, 128), dis_r, s_r, k == 2)

    f3 = s_r.reshape(NCORE, N, HALF)
    final = jnp.concatenate([f3[0], f3[1]], axis=1)
    return final[:N // 2], final[N // 2:]


# speedup vs baseline: 15.4999x; 15.4999x over previous
import functools

import jax
import jax.numpy as jnp
from jax import lax
from jax.experimental import pallas as pl
from jax.experimental.pallas import tpu as pltpu
from jax.experimental.pallas import tpu_sc as plsc

N = 50000
E = 800000
EMB = 64
QF = 16
NQ = 4
NSUB = 16
NCORE = 2
CHUNK = 128
GROUP = 8
ITERS = 49
EROWS = NSUB * ITERS * GROUP
E_PAD = EROWS * CHUNK
SUB_EROWS = ITERS * GROUP

NACC = 50048
DUMMY = 50040
ROWS_PER_SUB = NACC // NSUB

NROWS_R = NQ * NACC * QF // 128
BLK_R = 1088
GRID_R = NROWS_R // BLK_R


def _sc_mesh():
    return plsc.VectorSubcoreMesh(core_axis_name="c", subcore_axis_name="s")


def _edge_pass_body(with_gather, *refs):
    if with_gather:
        (row3, col3, y4, zeros, out,
         idx_r_v, idx_c_v, rows_v, acc, sem_i, sem_g, sem_s) = refs
    else:
        (col3, zeros, ones, out,
         idx_r_v, idx_c_v, rows_v, acc, sem_i, sem_g, sem_s) = refs
    c = lax.axis_index("c")
    s = lax.axis_index("s")

    if not with_gather:
        for j in range(GROUP):
            pltpu.sync_copy(ones, rows_v.at[j])

    n_passes = 2 if with_gather else 1
    for p in range(n_passes):
        q = 2 * c + p
        pltpu.sync_copy(zeros, acc.at[pl.ds(s * ROWS_PER_SUB, ROWS_PER_SUB)])
        plsc.subcore_barrier()

        def iter_body(g, carry):
            base = s * SUB_EROWS + g * GROUP
            if with_gather:
                cp_r = pltpu.async_copy(
                    row3.at[pl.ds(base, GROUP)], idx_r_v, sem_i)
            cp_c = pltpu.async_copy(col3.at[pl.ds(base, GROUP)], idx_c_v, sem_i)
            if with_gather:
                cp_r.wait()
                cp_c.wait()
                gathers = [
                    pltpu.async_copy(
                        y4.at[q].at[idx_r_v.at[j]], rows_v.at[j], sem_g)
                    for j in range(GROUP)
                ]
                for g_cp in gathers:
                    g_cp.wait()
            else:
                cp_c.wait()
            scatters = [
                pltpu.async_copy(
                    rows_v.at[j], acc.at[idx_c_v.at[j]], sem_s, add=True)
                for j in range(GROUP)
            ]
            for s_cp in scatters:
                s_cp.wait()
            return carry

        lax.fori_loop(0, ITERS, iter_body, 0)
        plsc.subcore_barrier()
        sl = pl.ds(s * ROWS_PER_SUB, ROWS_PER_SUB)
        if with_gather:
            pltpu.sync_copy(acc.at[sl], out.at[q].at[sl])
        else:
            pltpu.sync_copy(acc.at[sl], out.at[2 * c].at[sl])
            pltpu.sync_copy(acc.at[sl], out.at[2 * c + 1].at[sl])
        plsc.subcore_barrier()


def _make_edge_pass(with_gather):
    return pl.kernel(
        functools.partial(_edge_pass_body, with_gather),
        out_type=jax.ShapeDtypeStruct((NQ, NACC, QF), jnp.float32),
        mesh=_sc_mesh(),
        compiler_params=pltpu.CompilerParams(use_tc_tiling_on_sc=False),
        scratch_types=[
            pltpu.VMEM((GROUP, CHUNK), jnp.int32),
            pltpu.VMEM((GROUP, CHUNK), jnp.int32),
            pltpu.VMEM((GROUP, CHUNK, QF), jnp.float32),
            pltpu.VMEM_SHARED((NACC, QF), jnp.float32),
            pltpu.SemaphoreType.DMA,
            pltpu.SemaphoreType.DMA,
            pltpu.SemaphoreType.DMA,
        ],
    )


_sc_scatter = _make_edge_pass(True)
_sc_deg = _make_edge_pass(False)


def _tc_prep_body(deg_ref, x_ref, dis_ref, y_ref):
    deg = deg_ref[...]
    dis = jnp.where(deg > 0.0, lax.rsqrt(deg), 0.0)
    dis_ref[...] = dis
    y_ref[...] = dis * x_ref[...]


def _tc_prep(deg_r, x_r):
    spec = pl.BlockSpec((BLK_R, 128), lambda i: (i, 0))
    return pl.pallas_call(
        _tc_prep_body,
        grid=(GRID_R,),
        in_specs=[spec, spec],
        out_specs=[spec, spec],
        out_shape=[jax.ShapeDtypeStruct((NROWS_R, 128), jnp.float32)] * 2,
    )(deg_r, x_r)


def _tc_scale_body(is_final, z_ref, dis_ref, s_ref, so_ref, y_ref):
    dis = dis_ref[...]
    xk = dis * z_ref[...]
    snew = s_ref[...] + xk
    so_ref[...] = snew * 0.25 if is_final else snew
    y_ref[...] = dis * xk


def _tc_scale(z_r, dis_r, s_r, is_final):
    spec = pl.BlockSpec((BLK_R, 128), lambda i: (i, 0))
    return pl.pallas_call(
        functools.partial(_tc_scale_body, is_final),
        grid=(GRID_R,),
        in_specs=[spec, spec, spec],
        out_specs=[spec, spec],
        out_shape=[jax.ShapeDtypeStruct((NROWS_R, 128), jnp.float32)] * 2,
    )(z_r, dis_r, s_r)


def kernel(edge_index, user_emb, item_emb):
    x0 = jnp.concatenate([user_emb, item_emb], axis=0)
    x0p = jnp.pad(x0, ((0, NACC - N), (0, 0)))
    x0s = jnp.stack([x0p[:, q * QF:(q + 1) * QF] for q in range(NQ)])
    x0_r = x0s.reshape(NROWS_R, 128)
    pad_e = E_PAD - E
    row3 = jnp.concatenate(
        [edge_index[0], jnp.zeros((pad_e,), jnp.int32)]).reshape(EROWS, CHUNK)
    col3 = jnp.concatenate(
        [edge_index[1], jnp.full((pad_e,), DUMMY, jnp.int32)]).reshape(EROWS, CHUNK)
    zeros = jnp.zeros((ROWS_PER_SUB, QF), jnp.float32)
    ones = jnp.ones((CHUNK, QF), jnp.float32)

    deg4 = _sc_deg(col3, zeros, ones)
    dis_r, y_r = _tc_prep(deg4.reshape(NROWS_R, 128), x0_r)
    s_r = x0_r
    for k in range(3):
        z4 = _sc_scatter(row3, col3, y_r.reshape(NQ, NACC, QF), zeros)
        s_r, y_r = _tc_scale(z4.reshape(NROWS_R, 128), dis_r, s_r, k == 2)

    f4 = s_r.reshape(NQ, NACC, QF)[:, :N, :]
    final = jnp.concatenate([f4[q] for q in range(NQ)], axis=1)
    return final[:N // 2], final[N // 2:]
